# baseline (speedup 1.0000x reference)
import functools

import jax
import jax.numpy as jnp
from jax import lax
from jax.experimental import pallas as pl
from jax.experimental.pallas import tpu as pltpu
from jax.experimental.pallas import tpu_sc as plsc

BATCH = 16384
VOCAB = 32
D = 32
NCOL = 8
OUT_W = NCOL * D
NW = 32
NSUBC = 16

B_SC = 8192
B_TC = BATCH - B_SC
ROWS_W = B_SC // NW
CH = 128
NCH = ROWS_W // CH
CHW = CH * OUT_W
SMW = NCOL * CH
TAB_W = NCOL * VOCAB * D
BM = 512


def _sc_body(c0, c1, c2, c3, c4, c5, c6, c7, wtab, out,
             tab_v, ish, sm, buf_a, buf_b,
             sem_t, sem_i, sem_a, sem_b):
  cs = (c0, c1, c2, c3, c4, c5, c6, c7)
  cid = lax.axis_index("c")
  sid = lax.axis_index("s")
  wid = sid * 2 + cid
  b0 = wid * ROWS_W

  tab_cp = pltpu.async_copy(wtab, tab_v, sem_t)
  icopies = [
      pltpu.async_copy(
          cs[j].at[pl.ds(wid * NCH, NCH)],
          ish.at[sid, :, pl.ds(j * CH, CH)], sem_i)
      for j in range(NCOL)
  ]
  for cp in icopies:
    cp.wait()
  tab_cp.wait()

  bufs = (buf_a, buf_b)
  sem_o = (sem_a, sem_b)

  def fill(k, buf):
    def row(r, carry):
      dst = pl.multiple_of(r * OUT_W, OUT_W)
      for j in range(NCOL):
        c = sm[j * CH + r]
        base = pl.multiple_of(c * D + j * (VOCAB * D), D)
        for h in (0, 16):
          buf[pl.ds(dst + j * D + h, 16)] = tab_v[pl.ds(base + h, 16)]
      return carry

    lax.fori_loop(0, CH, row, 0)

  flushes = [None] * NCH
  for k in range(NCH):
    p = k % 2
    pltpu.sync_copy(ish.at[sid, k], sm)
    if k >= 2:
      flushes[k - 2].wait()
    fill(k, bufs[p])
    flushes[k] = pltpu.async_copy(
        bufs[p], out.at[pl.ds((b0 + k * CH) * OUT_W, CHW)], sem_o[p])

  for fl in flushes[-2:]:
    fl.wait()


def _tc_body(full, c0, c1, c2, c3, c4, c5, c6, c7,
             w0, w1, w2, w3, w4, w5, w6, w7, out):
  del full
  cs = (c0, c1, c2, c3, c4, c5, c6, c7)
  ws = (w0, w1, w2, w3, w4, w5, w6, w7)
  col = lax.broadcasted_iota(jnp.int32, (BM, VOCAB), 1)
  outs = []
  for j in range(NCOL):
    c = cs[j][0, 0, :]
    onehot = (c[:, None] == col).astype(jnp.float32)
    outs.append(
        jnp.dot(onehot, ws[j][...], preferred_element_type=jnp.float32))
  out[...] = jnp.concatenate(outs, axis=1)


@jax.jit
def _run(c0, c1, c2, c3, c4, c5, c6, c7,
         W_c0, W_c1, W_c2, W_c3, W_c4, W_c5, W_c6, W_c7):
  cs = (c0, c1, c2, c3, c4, c5, c6, c7)
  tabs = (W_c0, W_c1, W_c2, W_c3, W_c4, W_c5, W_c6, W_c7)

  wtab = jnp.concatenate(tabs, axis=0).reshape(TAB_W)
  cs_sc = [c[:B_SC].reshape(B_SC // CH, CH) for c in cs]
  mesh = plsc.VectorSubcoreMesh(core_axis_name="c", subcore_axis_name="s")
  sc_fn = pl.kernel(
      _sc_body,
      out_type=jax.ShapeDtypeStruct((BATCH * OUT_W,), jnp.float32),
      mesh=mesh,
      compiler_params=pltpu.CompilerParams(needs_layout_passes=False),
      scratch_types=[
          pltpu.VMEM((TAB_W,), jnp.float32),
          pltpu.VMEM_SHARED((NSUBC, NCH, SMW), jnp.int32),
          pltpu.SMEM((SMW,), jnp.int32),
          pltpu.VMEM((CHW,), jnp.float32),
          pltpu.VMEM((CHW,), jnp.float32),
          pltpu.SemaphoreType.DMA,
          pltpu.SemaphoreType.DMA,
          pltpu.SemaphoreType.DMA,
          pltpu.SemaphoreType.DMA,
      ],
  )
  sc_out = sc_fn(*cs_sc, wtab).reshape(BATCH, OUT_W)

  cs_tc = [c[B_SC:].reshape(B_TC // BM, 1, BM) for c in cs]
  tc_fn = pl.pallas_call(
      _tc_body,
      grid=(B_TC // BM,),
      in_specs=[pl.BlockSpec(memory_space=pl.ANY)]
      + [pl.BlockSpec((1, 1, BM), lambda i: (i, 0, 0))] * NCOL
      + [pl.BlockSpec((VOCAB, D), lambda i: (0, 0))] * NCOL,
      out_specs=pl.BlockSpec((BM, OUT_W), lambda i: (B_SC // BM + i, 0)),
      out_shape=jax.ShapeDtypeStruct((BATCH, OUT_W), jnp.float32),
      input_output_aliases={0: 0},
  )
  return tc_fn(sc_out, *cs_tc, *tabs)


def kernel(c0, c1, c2, c3, c4, c5, c6, c7,
           W_c0, W_c1, W_c2, W_c3, W_c4, W_c5, W_c6, W_c7):
  return _run(c0, c1, c2, c3, c4, c5, c6, c7,
              W_c0, W_c1, W_c2, W_c3, W_c4, W_c5, W_c6, W_c7)


# speedup vs baseline: 1.1383x; 1.1383x over previous
import functools

import jax
import jax.numpy as jnp
from jax import lax
from jax.experimental import pallas as pl
from jax.experimental.pallas import tpu as pltpu
from jax.experimental.pallas import tpu_sc as plsc

BATCH = 16384
VOCAB = 32
D = 32
NCOL = 8
OUT_W = NCOL * D
NW = 32
NSUBC = 16
ROWS_W = BATCH // NW
CH = 64
NCH = ROWS_W // CH
CHW = CH * OUT_W
SMW = NCOL * CH
IDX_W = NCOL * ROWS_W
TAB_W = NCOL * VOCAB * D


def _body(c0, c1, c2, c3, c4, c5, c6, c7, wtab, out,
          tab_v, ish, sm_a, sm_b, buf_a, buf_b,
          sem_t, sem_i, sem_sa, sem_sb, sem_a, sem_b):
  cs = (c0, c1, c2, c3, c4, c5, c6, c7)
  cid = lax.axis_index("c")
  sid = lax.axis_index("s")
  wid = sid * 2 + cid
  b0 = wid * ROWS_W

  tab_cp = pltpu.async_copy(wtab, tab_v, sem_t)
  icopies = [
      pltpu.async_copy(
          cs[j].at[pl.ds(b0, ROWS_W)],
          ish.at[sid, pl.ds(j * ROWS_W, ROWS_W)], sem_i)
      for j in range(NCOL)
  ]

  def stage_idx(k, sm, sem):
    return [
        pltpu.async_copy(
            ish.at[sid, pl.ds(j * ROWS_W + k * CH, CH)],
            sm.at[pl.ds(j * CH, CH)], sem)
        for j in range(NCOL)
    ]

  for cp in icopies:
    cp.wait()

  sms = (sm_a, sm_b)
  sem_s = (sem_sa, sem_sb)
  bufs = (buf_a, buf_b)
  sem_o = (sem_a, sem_b)

  stages = [None] * (NCH + 1)
  flushes = [None] * NCH
  stages[0] = stage_idx(0, sms[0], sem_s[0])
  tab_cp.wait()

  def fill(k, sm, buf):
    def row(r, carry):
      dst = pl.multiple_of(r * OUT_W, OUT_W)
      for j in range(NCOL):
        c = sm[j * CH + r]
        base = pl.multiple_of(c * D + j * (VOCAB * D), D)
        for h in (0, 16):
          buf[pl.ds(dst + j * D + h, 16)] = tab_v[pl.ds(base + h, 16)]
      return carry

    lax.fori_loop(0, CH, row, 0)

  for k in range(NCH):
    p = k % 2
    if k + 1 < NCH:
      stages[k + 1] = stage_idx(k + 1, sms[1 - p], sem_s[1 - p])
    for cp in stages[k]:
      cp.wait()
    if k >= 2:
      flushes[k - 2].wait()
    fill(k, sms[p], bufs[p])
    flushes[k] = pltpu.async_copy(
        bufs[p], out.at[pl.ds((b0 + k * CH) * OUT_W, CHW)], sem_o[p])

  flushes[NCH - 2].wait()
  flushes[NCH - 1].wait()


@jax.jit
def _run(c0, c1, c2, c3, c4, c5, c6, c7, wtab):
  mesh = plsc.VectorSubcoreMesh(core_axis_name="c", subcore_axis_name="s")
  f = pl.kernel(
      _body,
      out_type=jax.ShapeDtypeStruct((BATCH * OUT_W,), jnp.float32),
      mesh=mesh,
      compiler_params=pltpu.CompilerParams(needs_layout_passes=False),
      scratch_types=[
          pltpu.VMEM((TAB_W,), jnp.float32),
          pltpu.VMEM_SHARED((NSUBC, IDX_W), jnp.int32),
          pltpu.SMEM((SMW,), jnp.int32),
          pltpu.SMEM((SMW,), jnp.int32),
          pltpu.VMEM((CHW,), jnp.float32),
          pltpu.VMEM((CHW,), jnp.float32),
          pltpu.SemaphoreType.DMA,
          pltpu.SemaphoreType.DMA,
          pltpu.SemaphoreType.DMA,
          pltpu.SemaphoreType.DMA,
          pltpu.SemaphoreType.DMA,
          pltpu.SemaphoreType.DMA,
      ],
  )
  flat = f(c0, c1, c2, c3, c4, c5, c6, c7, wtab)
  return flat.reshape(BATCH, OUT_W)


def kernel(c0, c1, c2, c3, c4, c5, c6, c7,
           W_c0, W_c1, W_c2, W_c3, W_c4, W_c5, W_c6, W_c7):
  wtab = jnp.concatenate(
      [W_c0, W_c1, W_c2, W_c3, W_c4, W_c5, W_c6, W_c7], axis=0).reshape(TAB_W)
  return _run(c0, c1, c2, c3, c4, c5, c6, c7, wtab)
